# Initial kernel scaffold; baseline (speedup 1.0000x reference)
import functools

import jax
import jax.numpy as jnp
from jax import lax
from jax.experimental import pallas as pl
from jax.experimental.pallas import tpu as pltpu
from jax.experimental.pallas import tpu_sc as plsc

D = 64
B = 4096 * 50
NC, NS = 2, 16
NW = NC * NS
BPW = B // NW
CHUNK = 128
NCHUNK = BPW // CHUNK

_mesh = plsc.VectorSubcoreMesh(core_axis_name="c", subcore_axis_name="s")


@functools.partial(
    pl.kernel,
    mesh=_mesh,
    out_type=jax.ShapeDtypeStruct((B, D), jnp.float32),
    scratch_types=[
        pltpu.VMEM((NCHUNK, CHUNK), jnp.int32),
        pltpu.VMEM((CHUNK, D), jnp.float32),
        pltpu.SemaphoreType.DMA,
    ],
)
def _emb_lookup(idx_hbm, table_hbm, out_hbm, idx_v, rows_v, sem):
    wid = lax.axis_index("s") * NC + lax.axis_index("c")
    base = wid * BPW
    pltpu.sync_copy(idx_hbm.at[wid], idx_v)

    def body(j, carry):
        pltpu.async_copy(table_hbm.at[idx_v.at[j]], rows_v, sem).wait()
        pltpu.sync_copy(rows_v, out_hbm.at[pl.ds(base + j * CHUNK, CHUNK)])
        return carry

    lax.fori_loop(0, NCHUNK, body, 0)


def kernel(inputs, table):
    idx = inputs.reshape(NW, NCHUNK, CHUNK).astype(jnp.int32)
    out = _emb_lookup(idx, table)
    return out.reshape(inputs.shape[0], inputs.shape[1], D)


# speedup vs baseline: 4.0823x; 4.0823x over previous
import functools

import jax
import jax.numpy as jnp
from jax import lax
from jax.experimental import pallas as pl
from jax.experimental.pallas import tpu as pltpu
from jax.experimental.pallas import tpu_sc as plsc

D = 64
B = 4096 * 50
NC, NS = 2, 16
NW = NC * NS
BPW = B // NW
CHUNK = 128
NCHUNK = BPW // CHUNK

_mesh = plsc.VectorSubcoreMesh(core_axis_name="c", subcore_axis_name="s")


@functools.partial(
    pl.kernel,
    mesh=_mesh,
    out_type=jax.ShapeDtypeStruct((B, D), jnp.float32),
    compiler_params=pltpu.CompilerParams(use_tc_tiling_on_sc=False),
    scratch_types=[
        pltpu.VMEM((NCHUNK, CHUNK), jnp.int32),
        pltpu.VMEM((CHUNK, D), jnp.float32),
        pltpu.SemaphoreType.DMA,
    ],
)
def _emb_lookup(idx_hbm, table_hbm, out_hbm, idx_v, rows_v, sem):
    wid = lax.axis_index("s") * NC + lax.axis_index("c")
    base = wid * BPW
    pltpu.sync_copy(idx_hbm.at[wid], idx_v)

    def body(j, carry):
        pltpu.async_copy(table_hbm.at[idx_v.at[j]], rows_v, sem).wait()
        pltpu.sync_copy(rows_v, out_hbm.at[pl.ds(base + j * CHUNK, CHUNK)])
        return carry

    lax.fori_loop(0, NCHUNK, body, 0)


def kernel(inputs, table):
    idx = inputs.reshape(NW, NCHUNK, CHUNK).astype(jnp.int32)
    out = _emb_lookup(idx, table)
    return out.reshape(inputs.shape[0], inputs.shape[1], D)


# speedup vs baseline: 4.6152x; 1.1305x over previous
import functools

import jax
import jax.numpy as jnp
from jax import lax
from jax.experimental import pallas as pl
from jax.experimental.pallas import tpu as pltpu
from jax.experimental.pallas import tpu_sc as plsc

D = 64
B = 4096 * 50
NC, NS = 2, 16
NW = NC * NS
BPW = B // NW
CHUNK = 128
NCHUNK = BPW // CHUNK
NBUF = 5
NG = NCHUNK // NBUF
NSG = NG // 2

_mesh = plsc.VectorSubcoreMesh(core_axis_name="c", subcore_axis_name="s")


@functools.partial(
    pl.kernel,
    mesh=_mesh,
    out_type=jax.ShapeDtypeStruct((B, D), jnp.float32),
    compiler_params=pltpu.CompilerParams(use_tc_tiling_on_sc=False),
    scratch_types=[
        pltpu.VMEM((NCHUNK, CHUNK), jnp.int32),
        pltpu.VMEM((2, NBUF, CHUNK, D), jnp.float32),
        pltpu.SemaphoreType.DMA,
        pltpu.SemaphoreType.DMA,
        pltpu.SemaphoreType.DMA,
        pltpu.SemaphoreType.DMA,
    ],
)
def _emb_lookup(idx_hbm, table_hbm, out_hbm, idx_v, rows_v, gs0, gs1, ws0, ws1):
    gs = (gs0, gs1)
    ws = (ws0, ws1)
    wid = lax.axis_index("s") * NC + lax.axis_index("c")
    base = wid * BPW
    pltpu.sync_copy(idx_hbm.at[wid], idx_v)

    def gather(j, h, b):
        pltpu.async_copy(table_hbm.at[idx_v.at[j]], rows_v.at[h, b], gs[h])

    def wait_gathers(h):
        for b in range(NBUF):
            pltpu.make_async_copy(
                out_hbm.at[pl.ds(0, CHUNK)], rows_v.at[h, b], gs[h]
            ).wait()

    def drain_writes(h):
        for b in range(NBUF):
            pltpu.make_async_copy(
                out_hbm.at[pl.ds(0, CHUNK)], rows_v.at[h, b], ws[h]
            ).wait()

    for b in range(NBUF):
        gather(b, 0, b)

    def super_body(sg, carry):
        for h in (0, 1):
            g = 2 * sg + h
            oh = 1 - h
            wait_gathers(h)
            for b in range(NBUF):
                pltpu.async_copy(
                    rows_v.at[h, b],
                    out_hbm.at[pl.ds(base + (g * NBUF + b) * CHUNK, CHUNK)],
                    ws[h],
                )
            if h == 0:
                pl.when(sg > 0)(lambda: drain_writes(oh))
                for b in range(NBUF):
                    gather((g + 1) * NBUF + b, oh, b)
            else:
                drain_writes(oh)

                def refill():
                    for b in range(NBUF):
                        gather((g + 1) * NBUF + b, oh, b)

                pl.when(sg < NSG - 1)(refill)
        return carry

    lax.fori_loop(0, NSG, super_body, 0)
    drain_writes(1)


def kernel(inputs, table):
    idx = inputs.reshape(NW, NCHUNK, CHUNK).astype(jnp.int32)
    out = _emb_lookup(idx, table)
    return out.reshape(inputs.shape[0], inputs.shape[1], D)


# speedup vs baseline: 4.6736x; 1.0126x over previous
import functools

import jax
import jax.numpy as jnp
from jax import lax
from jax.experimental import pallas as pl
from jax.experimental.pallas import tpu as pltpu
from jax.experimental.pallas import tpu_sc as plsc

D = 64
B = 4096 * 50
NC, NS = 2, 16
NW = NC * NS
BPW = B // NW
CHUNK = 128
NCHUNK = BPW // CHUNK
K = 10
G = 5
NIT = NCHUNK // K

_mesh = plsc.VectorSubcoreMesh(core_axis_name="c", subcore_axis_name="s")


@functools.partial(
    pl.kernel,
    mesh=_mesh,
    out_type=jax.ShapeDtypeStruct((B, D), jnp.float32),
    compiler_params=pltpu.CompilerParams(use_tc_tiling_on_sc=False),
    scratch_types=[
        pltpu.VMEM((NCHUNK, CHUNK), jnp.int32),
        pltpu.VMEM((K, CHUNK, D), jnp.float32),
        pltpu.SemaphoreType.DMA((K,)),
        pltpu.SemaphoreType.DMA((K,)),
    ],
)
def _emb_lookup(idx_hbm, table_hbm, out_hbm, idx_v, rows_v, gsem, wsem):
    wid = lax.axis_index("s") * NC + lax.axis_index("c")
    base = wid * BPW
    pltpu.sync_copy(idx_hbm.at[wid], idx_v)

    def gather(j, b):
        pltpu.async_copy(table_hbm.at[idx_v.at[j]], rows_v.at[b], gsem.at[b])

    def wait_gather(b):
        pltpu.make_async_copy(
            out_hbm.at[pl.ds(0, CHUNK)], rows_v.at[b], gsem.at[b]
        ).wait()

    def drain_write(b):
        pltpu.make_async_copy(
            out_hbm.at[pl.ds(0, CHUNK)], rows_v.at[b], wsem.at[b]
        ).wait()

    for u in range(G):
        gather(u, u)

    def body(it, carry):
        j0 = it * K
        for u in range(K):
            j = j0 + u
            wait_gather(u)
            pltpu.async_copy(
                rows_v.at[u],
                out_hbm.at[pl.ds(base + j * CHUNK, CHUNK)],
                wsem.at[u],
            )
            bf = (u + G) % K

            def refill():
                gather(j + G, bf)

            def drain_and_refill():
                drain_write(bf)
                gather(j + G, bf)

            if u < G:
                pl.when(it > 0)(drain_and_refill)
                pl.when(it == 0)(refill)
            else:
                pl.when(it < NIT - 1)(drain_and_refill)
        return carry

    lax.fori_loop(0, NIT, body, 0)
    for u in range(K):
        drain_write(u)


def kernel(inputs, table):
    idx = inputs.reshape(NW, NCHUNK, CHUNK).astype(jnp.int32)
    out = _emb_lookup(idx, table)
    return out.reshape(inputs.shape[0], inputs.shape[1], D)
